# Initial kernel scaffold; baseline (speedup 1.0000x reference)
import jax
import jax.numpy as jnp
from jax.experimental import pallas as pl


def kernel(x, adj, W1):
    raise NotImplementedError("write your pallas kernel here")


# speedup vs baseline: 1.0401x; 1.0401x over previous
import jax
import jax.numpy as jnp
from jax.experimental import pallas as pl
from jax.experimental.pallas import tpu as pltpu

_N = 10000
_D = 128
_BM = 400


def _gcn_body(x_ref, adj_ref, w_ref, out_ref, support_ref):
    m = pl.program_id(0)

    @pl.when(m == 0)
    def _compute_support():
        support_ref[...] = jnp.dot(
            x_ref[...], w_ref[...], preferred_element_type=jnp.float32
        )

    x1 = jnp.dot(adj_ref[...], support_ref[...],
                 preferred_element_type=jnp.float32)
    x_blk = x_ref[pl.ds(m * _BM, _BM), :]
    out_ref[...] = 0.95 * x_blk + 0.05 * x1


def kernel(x, adj, W1):
    grid = (_N // _BM,)
    return pl.pallas_call(
        _gcn_body,
        grid=grid,
        in_specs=[
            pl.BlockSpec((_N, _D), lambda m: (0, 0)),
            pl.BlockSpec((_BM, _N), lambda m: (m, 0)),
            pl.BlockSpec((_D, _D), lambda m: (0, 0)),
        ],
        out_specs=pl.BlockSpec((_BM, _D), lambda m: (m, 0)),
        out_shape=jax.ShapeDtypeStruct((_N, _D), jnp.float32),
        scratch_shapes=[pltpu.VMEM((_N, _D), jnp.float32)],
    )(x, adj, W1)


# speedup vs baseline: 1.0426x; 1.0024x over previous
import jax
import jax.numpy as jnp
from jax.experimental import pallas as pl
from jax.experimental.pallas import tpu as pltpu

_N = 10000
_D = 128
_BM = 400


def _gcn_body(x_ref, adj_ref, w_ref, out_ref, support_ref):
    m = pl.program_id(0)

    @pl.when(m == 0)
    def _compute_support():
        support_ref[...] = jnp.dot(
            x_ref[...], w_ref[...], preferred_element_type=jnp.float32
        ).astype(jnp.bfloat16)

    x1 = jnp.dot(adj_ref[...].astype(jnp.bfloat16), support_ref[...],
                 preferred_element_type=jnp.float32)
    x_blk = x_ref[pl.ds(m * _BM, _BM), :]
    out_ref[...] = 0.95 * x_blk + 0.05 * x1


def kernel(x, adj, W1):
    grid = (_N // _BM,)
    return pl.pallas_call(
        _gcn_body,
        grid=grid,
        in_specs=[
            pl.BlockSpec((_N, _D), lambda m: (0, 0)),
            pl.BlockSpec((_BM, _N), lambda m: (m, 0)),
            pl.BlockSpec((_D, _D), lambda m: (0, 0)),
        ],
        out_specs=pl.BlockSpec((_BM, _D), lambda m: (m, 0)),
        out_shape=jax.ShapeDtypeStruct((_N, _D), jnp.float32),
        scratch_shapes=[pltpu.VMEM((_N, _D), jnp.bfloat16)],
    )(x, adj, W1)
